# Initial kernel scaffold; baseline (speedup 1.0000x reference)
import jax
import jax.numpy as jnp
from jax.experimental import pallas as pl


def kernel(write_weights, free_gate, read_weights, prev_usage):
    raise NotImplementedError("write your pallas kernel here")


# speedup vs baseline: 2.2905x; 2.2905x over previous
import jax
import jax.numpy as jnp
from jax.experimental import pallas as pl
from jax.experimental.pallas import tpu as pltpu

B = 1024
NUM_WRITES = 2
NUM_READS = 4
MEMORY_SIZE = 8192

ROWS = 16


def _body(ww_ref, fg_ref, rw_ref, prev_ref, out_ref):
    prev = prev_ref[...]
    q = (1.0 - ww_ref[:, 0, :]) * (1.0 - ww_ref[:, 1, :])
    usage = 1.0 - (1.0 - prev) * q
    fg = fg_ref[...]
    phi = (1.0 - fg[:, 0, None] * rw_ref[:, 0, :])
    phi = phi * (1.0 - fg[:, 1, None] * rw_ref[:, 1, :])
    phi = phi * (1.0 - fg[:, 2, None] * rw_ref[:, 2, :])
    phi = phi * (1.0 - fg[:, 3, None] * rw_ref[:, 3, :])
    out_ref[...] = usage * phi


def kernel(write_weights, free_gate, read_weights, prev_usage):
    grid = (B // ROWS,)
    return pl.pallas_call(
        _body,
        grid=grid,
        in_specs=[
            pl.BlockSpec((ROWS, NUM_WRITES, MEMORY_SIZE), lambda i: (i, 0, 0)),
            pl.BlockSpec((ROWS, NUM_READS), lambda i: (i, 0)),
            pl.BlockSpec((ROWS, NUM_READS, MEMORY_SIZE), lambda i: (i, 0, 0)),
            pl.BlockSpec((ROWS, MEMORY_SIZE), lambda i: (i, 0)),
        ],
        out_specs=pl.BlockSpec((ROWS, MEMORY_SIZE), lambda i: (i, 0)),
        out_shape=jax.ShapeDtypeStruct((B, MEMORY_SIZE), jnp.float32),
        compiler_params=pltpu.CompilerParams(
            dimension_semantics=("arbitrary",),
        ),
    )(write_weights, free_gate, read_weights, prev_usage)


# speedup vs baseline: 2.5745x; 1.1240x over previous
import jax
import jax.numpy as jnp
from jax.experimental import pallas as pl
from jax.experimental.pallas import tpu as pltpu

B = 1024
NUM_WRITES = 2
NUM_READS = 4
MEMORY_SIZE = 8192

ROWS = 32


def _body(ww_ref, fg_ref, rw_ref, prev_ref, out_ref):
    prev = prev_ref[...]
    q = (1.0 - ww_ref[:, 0, :]) * (1.0 - ww_ref[:, 1, :])
    usage = 1.0 - (1.0 - prev) * q
    fg = fg_ref[...]
    phi = (1.0 - fg[:, 0, None] * rw_ref[:, 0, :])
    phi = phi * (1.0 - fg[:, 1, None] * rw_ref[:, 1, :])
    phi = phi * (1.0 - fg[:, 2, None] * rw_ref[:, 2, :])
    phi = phi * (1.0 - fg[:, 3, None] * rw_ref[:, 3, :])
    out_ref[...] = usage * phi


def kernel(write_weights, free_gate, read_weights, prev_usage):
    grid = (B // ROWS,)
    return pl.pallas_call(
        _body,
        grid=grid,
        in_specs=[
            pl.BlockSpec((ROWS, NUM_WRITES, MEMORY_SIZE), lambda i: (i, 0, 0)),
            pl.BlockSpec((ROWS, NUM_READS), lambda i: (i, 0)),
            pl.BlockSpec((ROWS, NUM_READS, MEMORY_SIZE), lambda i: (i, 0, 0)),
            pl.BlockSpec((ROWS, MEMORY_SIZE), lambda i: (i, 0)),
        ],
        out_specs=pl.BlockSpec((ROWS, MEMORY_SIZE), lambda i: (i, 0)),
        out_shape=jax.ShapeDtypeStruct((B, MEMORY_SIZE), jnp.float32),
        compiler_params=pltpu.CompilerParams(
            dimension_semantics=("arbitrary",),
        ),
    )(write_weights, free_gate, read_weights, prev_usage)


# speedup vs baseline: 2.7770x; 1.0786x over previous
import jax
import jax.numpy as jnp
from jax.experimental import pallas as pl
from jax.experimental.pallas import tpu as pltpu

B = 1024
NUM_WRITES = 2
NUM_READS = 4
MEMORY_SIZE = 8192

ROWS = 64


def _body(ww_ref, fg_ref, rw_ref, prev_ref, out_ref):
    prev = prev_ref[...]
    q = (1.0 - ww_ref[:, 0, :]) * (1.0 - ww_ref[:, 1, :])
    usage = 1.0 - (1.0 - prev) * q
    fg = fg_ref[...]
    phi = (1.0 - fg[:, 0, None] * rw_ref[:, 0, :])
    phi = phi * (1.0 - fg[:, 1, None] * rw_ref[:, 1, :])
    phi = phi * (1.0 - fg[:, 2, None] * rw_ref[:, 2, :])
    phi = phi * (1.0 - fg[:, 3, None] * rw_ref[:, 3, :])
    out_ref[...] = usage * phi


def kernel(write_weights, free_gate, read_weights, prev_usage):
    grid = (B // ROWS,)
    return pl.pallas_call(
        _body,
        grid=grid,
        in_specs=[
            pl.BlockSpec((ROWS, NUM_WRITES, MEMORY_SIZE), lambda i: (i, 0, 0)),
            pl.BlockSpec((ROWS, NUM_READS), lambda i: (i, 0)),
            pl.BlockSpec((ROWS, NUM_READS, MEMORY_SIZE), lambda i: (i, 0, 0)),
            pl.BlockSpec((ROWS, MEMORY_SIZE), lambda i: (i, 0)),
        ],
        out_specs=pl.BlockSpec((ROWS, MEMORY_SIZE), lambda i: (i, 0)),
        out_shape=jax.ShapeDtypeStruct((B, MEMORY_SIZE), jnp.float32),
        compiler_params=pltpu.CompilerParams(
            dimension_semantics=("arbitrary",),
        ),
    )(write_weights, free_gate, read_weights, prev_usage)
